# Initial kernel scaffold; baseline (speedup 1.0000x reference)
import jax
import jax.numpy as jnp
from jax.experimental import pallas as pl


def kernel(inputs):
    raise NotImplementedError("write your pallas kernel here")


# speedup vs baseline: 2.1514x; 2.1514x over previous
import jax
import jax.numpy as jnp
from jax.experimental import pallas as pl

S = 8192
K = 64
BC = 128


def _rev_axis1(x):
    N, L, C = x.shape
    d = L // 2
    while d >= 1:
        y = x.reshape(-1, 2, d, C)
        x = jnp.concatenate([y[:, 1], y[:, 0]], axis=1).reshape(N, L, C)
        d //= 2
    return x


def _merge_down(x):
    N, L, C = x.shape
    d = L // 2
    while d >= 1:
        y = x.reshape(-1, 2, d, C)
        a, b = y[:, 0], y[:, 1]
        mx = jnp.maximum(a, b)
        mn = jnp.minimum(a, b)
        x = jnp.concatenate([mx, mn], axis=1).reshape(N, L, C)
        d //= 2
    return x


def _sort_desc_runs(x2d, run_len):
    _, C = x2d.shape
    x = x2d.reshape(-1, 1, C)
    L = 1
    while L < run_len:
        y = x.reshape(-1, 2, L, C)
        a, b = y[:, 0], y[:, 1]
        c = jnp.concatenate([a, _rev_axis1(b)], axis=1)
        x = _merge_down(c)
        L *= 2
    return x


def _topk_kernel(x_ref, o_ref):
    x = x_ref[0]
    cur = _sort_desc_runs(x, K)
    while cur.shape[0] > 1:
        y = cur.reshape(-1, 2, K, BC)
        a, b = y[:, 0], y[:, 1]
        m = jnp.maximum(a, _rev_axis1(b))
        cur = _merge_down(m)
    o_ref[0] = cur[0]


def kernel(inputs):
    B, s, C = inputs.shape
    assert s == S and C % BC == 0
    grid = (B, C // BC)
    return pl.pallas_call(
        _topk_kernel,
        grid=grid,
        in_specs=[pl.BlockSpec((1, S, BC), lambda b, c: (b, 0, c))],
        out_specs=pl.BlockSpec((1, K, BC), lambda b, c: (b, 0, c)),
        out_shape=jax.ShapeDtypeStruct((B, K, C), jnp.float32),
    )(inputs)


# speedup vs baseline: 26.2854x; 12.2178x over previous
import jax
import jax.numpy as jnp
from jax.experimental import pallas as pl

S = 8192
K = 64
BC = 128
G0 = S // K


def _rev1(x):
    N, L, M, C = x.shape
    d = L // 2
    while d >= 1:
        y = x.reshape(-1, 2, d, M, C)
        x = jnp.concatenate([y[:, 1], y[:, 0]], axis=1).reshape(N, L, M, C)
        d //= 2
    return x


def _merge_down1(x):
    N, L, M, C = x.shape
    d = L // 2
    while d >= 1:
        y = x.reshape(-1, 2, d, M, C)
        a, b = y[:, 0], y[:, 1]
        x = jnp.concatenate(
            [jnp.maximum(a, b), jnp.minimum(a, b)], axis=1
        ).reshape(N, L, M, C)
        d //= 2
    return x


def _topk_kernel(x_ref, o_ref):
    x = x_ref[0].reshape(K, 1, G0, BC)
    L = 1
    while L < K:
        y = x.reshape(-1, 2, L, G0, BC)
        a, b = y[:, 0], y[:, 1]
        x = _merge_down1(jnp.concatenate([a, _rev1(b)], axis=1))
        L *= 2
    cur = x
    g = G0
    while g > 1:
        h = g // 2
        a, b = cur[:, :, :h, :], cur[:, :, h:, :]
        m = jnp.maximum(a, _rev1(b))
        cur = _merge_down1(m)
        g = h
    o_ref[0] = cur[0, :, 0, :]


def kernel(inputs):
    B, s, C = inputs.shape
    assert s == S and C % BC == 0
    grid = (B, C // BC)
    return pl.pallas_call(
        _topk_kernel,
        grid=grid,
        in_specs=[pl.BlockSpec((1, S, BC), lambda b, c: (b, 0, c))],
        out_specs=pl.BlockSpec((1, K, BC), lambda b, c: (b, 0, c)),
        out_shape=jax.ShapeDtypeStruct((B, K, C), jnp.float32),
    )(inputs)
